# Initial kernel scaffold; baseline (speedup 1.0000x reference)
import jax
import jax.numpy as jnp
from jax.experimental import pallas as pl


def kernel(x):
    raise NotImplementedError("write your pallas kernel here")


# speedup vs baseline: 6.2876x; 6.2876x over previous
import jax
import jax.numpy as jnp
import numpy as np
from jax.experimental import pallas as pl
from jax.experimental.pallas import tpu as pltpu

BLK = 64
NBINS = 256
NTILES = 16
TILE_N = BLK * BLK


def _resize_weights() -> np.ndarray:
    i = np.arange(256, dtype=np.float64)
    src = (i + 0.5) / 64.0 - 0.5
    i0 = np.floor(src)
    f = src - i0
    a0 = np.clip(i0, 0, 3).astype(np.int64)
    a1 = np.clip(i0 + 1, 0, 3).astype(np.int64)
    w = np.zeros((256, 4), np.float64)
    w[np.arange(256), a0] += 1.0 - f
    w[np.arange(256), a1] += f
    return w.astype(np.float32)


def _basis() -> np.ndarray:
    w = _resize_weights()
    b = np.einsum("ia,jb->abij", w, w)
    return np.ascontiguousarray(b.reshape(16, 256, 256))


_BASIS = _basis()


def _channel_kernel(x_ref, b_ref, o_ref):
    binv = jax.lax.broadcasted_iota(jnp.int32, (NBINS, 128), 0).astype(jnp.float32)
    acc = jnp.zeros((256, 256), jnp.float32)
    for t in range(NTILES):
        d = x_ref[0, t]
        mn = jnp.min(d)
        mx = jnp.max(d)
        width = (mx - mn) / NBINS
        wsafe = jnp.where(width > 0, width, 1.0)
        idx = jnp.clip(jnp.floor((d - mn) / wsafe), 0.0, NBINS - 1.0)
        hacc = jnp.zeros((NBINS, 128), jnp.float32)
        for r in range(32):
            row = jax.lax.broadcast_in_dim(idx[r : r + 1, :], (NBINS, 128), (0, 1))
            hacc = hacc + (row == binv).astype(jnp.float32)
        counts = jnp.sum(hacc, axis=1, keepdims=True)
        dens = counts / (TILE_N * wsafe)
        p = dens + 1e-12
        p = p / jnp.sum(p)
        ent = -jnp.sum(p * (jnp.log(p) * np.float32(1.4426950408889634)))
        acc = acc + ent * b_ref[t]
    o_ref[0] = acc


def kernel(x):
    n, c, h, w = x.shape
    nc = n * c
    xr = (
        x.reshape(nc, 4, BLK, 4, BLK)
        .transpose(0, 1, 3, 2, 4)
        .reshape(nc, NTILES, 32, 128)
    )
    basis = jnp.asarray(_BASIS)
    out = pl.pallas_call(
        _channel_kernel,
        grid=(nc,),
        in_specs=[
            pl.BlockSpec((1, NTILES, 32, 128), lambda i: (i, 0, 0, 0)),
            pl.BlockSpec((NTILES, 256, 256), lambda i: (0, 0, 0)),
        ],
        out_specs=pl.BlockSpec((1, 256, 256), lambda i: (i, 0, 0)),
        out_shape=jax.ShapeDtypeStruct((nc, 256, 256), jnp.float32),
        compiler_params=pltpu.CompilerParams(
            dimension_semantics=("parallel",),
        ),
    )(xr, basis)
    return out.reshape(n, c, h, w)


# speedup vs baseline: 14.6600x; 2.3316x over previous
import jax
import jax.numpy as jnp
import numpy as np
from jax.experimental import pallas as pl
from jax.experimental.pallas import tpu as pltpu

BLK = 64
NBINS = 256
NTILES = 16
TILE_N = BLK * BLK
EPS = 1e-12


def _resize_weights() -> np.ndarray:
    i = np.arange(256, dtype=np.float64)
    src = (i + 0.5) / 64.0 - 0.5
    i0 = np.floor(src)
    f = src - i0
    a0 = np.clip(i0, 0, 3).astype(np.int64)
    a1 = np.clip(i0 + 1, 0, 3).astype(np.int64)
    w = np.zeros((256, 4), np.float64)
    w[np.arange(256), a0] += 1.0 - f
    w[np.arange(256), a1] += f
    return w.astype(np.float32)


def _binmap() -> np.ndarray:
    g = np.arange(16)[:, None, None]
    s = np.arange(16)[None, :, None]
    v = (16 * g + s).astype(np.int16)
    return np.broadcast_to(v, (16, 16, 128)).copy()


_W = _resize_weights()
_WVT = np.ascontiguousarray(_W.T)
_WHREP = np.ascontiguousarray(
    np.broadcast_to(_W.T[:, :, None], (4, 256, 128))
)
_BINMAP = _binmap()


def _channel_kernel(x_ref, wv_ref, wh_ref, m_ref, o_ref):
    one16 = jnp.int16(1)
    laneiota = jax.lax.broadcasted_iota(jnp.int32, (1, NTILES), 1)
    counts16 = jnp.zeros((NBINS, NTILES), jnp.float32)
    wvec = jnp.zeros((1, NTILES), jnp.float32)
    binmaps = [m_ref[g] for g in range(16)]
    for t in range(NTILES):
        d = x_ref[0, t]
        mn = jnp.min(d)
        mx = jnp.max(d)
        width = (mx - mn) / NBINS
        wsafe = jnp.where(width > 0, width, 1.0)
        idx = jnp.clip(jnp.floor((d - mn) / wsafe), 0.0, 255.0)
        hacc = [jnp.zeros((16, 128), jnp.int16) for _ in range(16)]
        for r in range(32):
            rb = jax.lax.broadcast_in_dim(idx[r : r + 1, :], (16, 128), (0, 1))
            rb16 = jnp.round(rb).astype(jnp.int16)
            for g in range(16):
                hacc[g] = jnp.where(rb16 == binmaps[g], hacc[g] + one16, hacc[g])
        hf = jnp.concatenate([h.astype(jnp.float32) for h in hacc], axis=0)
        cnt = jnp.sum(hf, axis=1, keepdims=True)
        sel = (laneiota == t).astype(jnp.float32)
        counts16 = counts16 + cnt * sel
        wvec = wvec + wsafe * sel
    wadd = (EPS * TILE_N) * wvec
    u = counts16 + wadd
    lg = jnp.log2(jnp.maximum(u, 1e-30))
    t1 = jnp.sum(u * lg, axis=0, keepdims=True)
    ucap = TILE_N + NBINS * wadd
    ent = jnp.log2(ucap) - t1 / ucap
    for k in range(2):
        och = jnp.zeros((256, 128), jnp.float32)
        for a in range(4):
            g = jnp.zeros((1, 128), jnp.float32)
            for b in range(4):
                g = g + ent[0, 4 * a + b] * wv_ref[b : b + 1, 128 * k : 128 * (k + 1)]
            gb = jax.lax.broadcast_in_dim(g, (256, 128), (0, 1))
            och = och + wh_ref[a] * gb
        o_ref[0, :, 128 * k : 128 * (k + 1)] = och


def kernel(x):
    n, c, h, w = x.shape
    nc = n * c
    xr = (
        x.reshape(nc, 4, BLK, 4, BLK)
        .transpose(0, 1, 3, 2, 4)
        .reshape(nc, NTILES, 32, 128)
    )
    out = pl.pallas_call(
        _channel_kernel,
        grid=(nc,),
        in_specs=[
            pl.BlockSpec((1, NTILES, 32, 128), lambda i: (i, 0, 0, 0)),
            pl.BlockSpec((4, 256), lambda i: (0, 0)),
            pl.BlockSpec((4, 256, 128), lambda i: (0, 0, 0)),
            pl.BlockSpec((16, 16, 128), lambda i: (0, 0, 0)),
        ],
        out_specs=pl.BlockSpec((1, 256, 256), lambda i: (i, 0, 0)),
        out_shape=jax.ShapeDtypeStruct((nc, 256, 256), jnp.float32),
        compiler_params=pltpu.CompilerParams(
            dimension_semantics=("arbitrary",),
        ),
    )(xr, jnp.asarray(_WVT), jnp.asarray(_WHREP), jnp.asarray(_BINMAP))
    return out.reshape(n, c, h, w)
